# baseline (speedup 1.0000x reference)
import jax
import jax.numpy as jnp
from jax import lax
from jax.experimental import pallas as pl
from jax.experimental.pallas import tpu as pltpu

_E = 64
_D = 1024
_N = 128
_K = 6
_S = 2


def _route_body(m_ref, ids_ref, n_ref):
    m1x = m_ref[...]
    e_iota = lax.broadcasted_iota(jnp.int32, (_E, _N), 0)
    eq = (e_iota == m1x).astype(jnp.int32)
    pres = jnp.max(eq, axis=1, keepdims=True)
    row = lax.broadcasted_iota(jnp.int32, (_E, _E), 0)
    col = lax.broadcasted_iota(jnp.int32, (_E, _E), 1)
    lower = (row > col).astype(jnp.float32)
    xrank = lax.dot_general(
        lower, pres.astype(jnp.float32),
        dimension_numbers=(((1,), (0,)), ((), ())),
        preferred_element_type=jnp.float32,
    ).astype(jnp.int32)
    slot = (xrank == col) & (pres > 0)
    slot_i = slot.astype(jnp.int32)
    ids = jnp.sum(slot_i * row, axis=0, keepdims=True)
    filled = jnp.sum(slot_i, axis=0, keepdims=True)
    eids = lax.broadcasted_iota(jnp.int32, (_E, 1), 0)
    maxid = jnp.max(pres * eids)
    ids_ref[...] = ids + (1 - filled) * maxid
    n_ref[...] = jnp.sum(pres).reshape(1, 1)


def _route(m2d):
    return pl.pallas_call(
        _route_body,
        out_shape=[
            jax.ShapeDtypeStruct((1, _E), jnp.int32),
            jax.ShapeDtypeStruct((1, 1), jnp.int32),
        ],
    )(m2d)


def _moe_body(ids_ref, m_ref, x_ref, *rest):
    w_refs = rest[:_K * _S]
    b_refs = rest[_K * _S:_K * _S + _K]
    o_ref = rest[_K * _S + _K]
    i = pl.program_id(0)

    @pl.when(i == 0)
    def _init():
        o_ref[...] = jnp.zeros_like(o_ref)

    x = x_ref[...]
    m = m_ref[...]
    acc = jnp.zeros((_N, _D), jnp.float32)
    prev_e = None
    for j in range(_K):
        e = ids_ref[_K * i + j]
        parts = [
            jax.lax.dot_general(
                x, w_refs[s * _K + j][0],
                dimension_numbers=(((1,), (1,)), ((), ())),
                preferred_element_type=jnp.float32,
            )
            for s in range(_S)
        ]
        xw = jnp.concatenate(parts, axis=1) + b_refs[j][0]
        mask = m == e if j == 0 else (m == e) & (e != prev_e)
        acc = acc + jnp.where(mask, xw, 0.0)
        prev_e = e
    o_ref[...] += acc


def kernel(x, module_indices, W, b):
    m = module_indices.astype(jnp.int32)
    ids2d, n2d = _route(m.reshape(1, _N))
    ids = ids2d.reshape(_E)
    n_used = n2d[0, 0]
    n_steps = (n_used + _K - 1) // _K

    m2d = m.reshape(_N, 1)
    b3 = b.reshape(_E, 1, _D)

    def w_spec(j, h):
        return pl.BlockSpec(
            (1, _D // _S, _D), lambda i, ids, j=j, h=h: (ids[_K * i + j], h, 0))

    def b_spec(j):
        return pl.BlockSpec(
            (1, 1, _D), lambda i, ids, j=j: (ids[_K * i + j], 0, 0))

    grid_spec = pltpu.PrefetchScalarGridSpec(
        num_scalar_prefetch=1,
        grid=(n_steps,),
        in_specs=[
            pl.BlockSpec((_N, 1), lambda i, ids: (0, 0)),
            pl.BlockSpec((_N, _D), lambda i, ids: (0, 0)),
            *[w_spec(j, s) for s in range(_S) for j in range(_K)],
            *[b_spec(j) for j in range(_K)],
        ],
        out_specs=pl.BlockSpec((_N, _D), lambda i, ids: (0, 0)),
    )

    out = pl.pallas_call(
        _moe_body,
        grid_spec=grid_spec,
        out_shape=jax.ShapeDtypeStruct((_N, _D), jnp.float32),
        compiler_params=pltpu.CompilerParams(
            dimension_semantics=("arbitrary",),
        ),
    )(ids, m2d, x, *([W] * (_S * _K)), *([b3] * _K))
    return out


# speedup vs baseline: 1.0927x; 1.0927x over previous
import jax
import jax.numpy as jnp
from jax import lax
from jax.experimental import pallas as pl
from jax.experimental.pallas import tpu as pltpu

_E = 64
_D = 1024
_N = 128
_K = 4
_S = 2


def _route_body(m_ref, ids_ref, n_ref):
    m1x = m_ref[...]
    e_iota = lax.broadcasted_iota(jnp.int32, (_E, _N), 0)
    eq = (e_iota == m1x).astype(jnp.int32)
    pres = jnp.max(eq, axis=1, keepdims=True)
    row = lax.broadcasted_iota(jnp.int32, (_E, _E), 0)
    col = lax.broadcasted_iota(jnp.int32, (_E, _E), 1)
    lower = (row > col).astype(jnp.float32)
    xrank = lax.dot_general(
        lower, pres.astype(jnp.float32),
        dimension_numbers=(((1,), (0,)), ((), ())),
        preferred_element_type=jnp.float32,
    ).astype(jnp.int32)
    slot = (xrank == col) & (pres > 0)
    slot_i = slot.astype(jnp.int32)
    ids = jnp.sum(slot_i * row, axis=0, keepdims=True)
    filled = jnp.sum(slot_i, axis=0, keepdims=True)
    eids = lax.broadcasted_iota(jnp.int32, (_E, 1), 0)
    maxid = jnp.max(pres * eids)
    ids_ref[...] = ids + (1 - filled) * maxid
    n_ref[...] = jnp.sum(pres).reshape(1, 1)


def _route(m2d):
    return pl.pallas_call(
        _route_body,
        out_shape=[
            jax.ShapeDtypeStruct((1, _E), jnp.int32),
            jax.ShapeDtypeStruct((1, 1), jnp.int32),
        ],
    )(m2d)


def _moe_body(ids_ref, m_ref, x_ref, *rest):
    w_refs = rest[:_K * _S]
    b_refs = rest[_K * _S:_K * _S + _K]
    o_ref = rest[_K * _S + _K]
    i = pl.program_id(0)

    @pl.when(i == 0)
    def _init():
        o_ref[...] = jnp.zeros_like(o_ref)

    x = x_ref[...]
    m = m_ref[...]
    acc = jnp.zeros((_N, _D), jnp.float32)
    prev_e = None
    for j in range(_K):
        e = ids_ref[_K * i + j]
        parts = [
            jax.lax.dot_general(
                x, w_refs[s * _K + j][0],
                dimension_numbers=(((1,), (1,)), ((), ())),
                preferred_element_type=jnp.float32,
            )
            for s in range(_S)
        ]
        xw = jnp.concatenate(parts, axis=1) + b_refs[j][0]
        mask = m == e if j == 0 else (m == e) & (e != prev_e)
        acc = acc + jnp.where(mask, xw, 0.0)
        prev_e = e
    o_ref[...] += acc


def kernel(x, module_indices, W, b):
    m = module_indices.astype(jnp.int32)
    ids2d, n2d = _route(m.reshape(1, _N))
    ids = ids2d.reshape(_E)
    n_used = n2d[0, 0]
    n_steps = (n_used + _K - 1) // _K

    m2d = m.reshape(_N, 1)
    b3 = b.reshape(_E, 1, _D)

    def w_spec(j, h):
        return pl.BlockSpec(
            (1, _D // _S, _D), lambda i, ids, j=j, h=h: (ids[_K * i + j], h, 0))

    def b_spec(j):
        return pl.BlockSpec(
            (1, 1, _D), lambda i, ids, j=j: (ids[_K * i + j], 0, 0))

    grid_spec = pltpu.PrefetchScalarGridSpec(
        num_scalar_prefetch=1,
        grid=(n_steps,),
        in_specs=[
            pl.BlockSpec((_N, 1), lambda i, ids: (0, 0)),
            pl.BlockSpec((_N, _D), lambda i, ids: (0, 0)),
            *[w_spec(j, s) for s in range(_S) for j in range(_K)],
            *[b_spec(j) for j in range(_K)],
        ],
        out_specs=pl.BlockSpec((_N, _D), lambda i, ids: (0, 0)),
    )

    out = pl.pallas_call(
        _moe_body,
        grid_spec=grid_spec,
        out_shape=jax.ShapeDtypeStruct((_N, _D), jnp.float32),
        compiler_params=pltpu.CompilerParams(
            dimension_semantics=("arbitrary",),
        ),
    )(ids, m2d, x, *([W] * (_S * _K)), *([b3] * _K))
    return out
